# baseline (speedup 1.0000x reference)
import functools

import jax
import jax.numpy as jnp
from jax import lax
from jax.experimental import pallas as pl
from jax.experimental.pallas import tpu as pltpu
from jax.experimental.pallas import tpu_sc as plsc

NUM_NODES = 100000
D = 128
B = 16384
M = 32

NC = 2
NS = 16
L = 16
NW = NC * NS
PER_W = B // NW
CH = 4
NCH = PER_W // CH
NG = D // L


def _splat(i32_scalar):
    return jnp.full((L,), i32_scalar, dtype=jnp.int32)


def _sc_body(table_hbm, idx_hbm, len_hbm, out_hbm,
             idx_raw, sel_v, len_v, recip_v, s2_v, rows_v, obuf, emb0_v, sem):
    wid = lax.axis_index("s") * NC + lax.axis_index("c")
    base = wid * PER_W

    pltpu.sync_copy(idx_hbm.at[pl.ds(base, PER_W)], idx_raw)
    pltpu.sync_copy(len_hbm.at[pl.ds(base, PER_W)], len_v)
    pltpu.sync_copy(table_hbm.at[0], emb0_v)

    iota = lax.iota(jnp.int32, L)

    def scales_body(g, _):
        l16 = len_v[pl.ds(g * L, L)]
        lenf = l16.astype(jnp.float32)
        rec = 1.0 / lenf
        recip_v[pl.ds(g * L, L)] = rec
        s2_v[pl.ds(g * L, L)] = (jnp.float32(M) - lenf) * rec
        return 0
    lax.fori_loop(0, PER_W // L, scales_body, 0, unroll=1)

    def mask_body(e, _):
        lsp = plsc.load_gather(len_v, [_splat(e)])
        iv0 = plsc.load_gather(idx_raw, [_splat(e), iota])
        iv1 = plsc.load_gather(idx_raw, [_splat(e), iota + L])
        s0 = jnp.where(iota < lsp, iv0, 0)
        s1 = jnp.where(iota + L < lsp, iv1, 0)
        row = e // CH
        col = (e % CH) * M
        plsc.store_scatter(sel_v, [_splat(row), col + iota], s0)
        plsc.store_scatter(sel_v, [_splat(row), col + L + iota], s1)
        return 0
    lax.fori_loop(0, PER_W, mask_body, 0, unroll=1)

    def chunk_body(c, _):
        pltpu.async_copy(table_hbm.at[sel_v.at[c]], rows_v, sem).wait()
        for el in range(CH):
            edge = c * CH + el
            rsp = plsc.load_gather(recip_v, [_splat(edge)])
            ssp = plsc.load_gather(s2_v, [_splat(edge)])
            r0 = el * M

            def acc_body(j, accs):
                return tuple(
                    accs[g] + rows_v[r0 + j, pl.ds(g * L, L)] for g in range(NG)
                )
            init = tuple(rows_v[r0, pl.ds(g * L, L)] for g in range(NG))
            accs = lax.fori_loop(1, M, acc_body, init, unroll=4)
            for g in range(NG):
                obuf[el, pl.ds(g * L, L)] = (
                    accs[g] * rsp - emb0_v[pl.ds(g * L, L)] * ssp
                )
        pltpu.sync_copy(obuf, out_hbm.at[pl.ds(base + c * CH, CH)])
        return 0
    lax.fori_loop(0, NCH, chunk_body, 0, unroll=1)


def kernel(node_embeddings, padded_hyperedges, hyperedge_lengths):
    mesh = plsc.VectorSubcoreMesh(core_axis_name="c", subcore_axis_name="s")
    f = pl.kernel(
        _sc_body,
        out_type=jax.ShapeDtypeStruct((B, D), jnp.float32),
        mesh=mesh,
        compiler_params=pltpu.CompilerParams(needs_layout_passes=False),
        scratch_types=[
            pltpu.VMEM((PER_W, M), jnp.int32),
            pltpu.VMEM((NCH, CH * M), jnp.int32),
            pltpu.VMEM((PER_W,), jnp.int32),
            pltpu.VMEM((PER_W,), jnp.float32),
            pltpu.VMEM((PER_W,), jnp.float32),
            pltpu.VMEM((CH * M, D), jnp.float32),
            pltpu.VMEM((CH, D), jnp.float32),
            pltpu.VMEM((D,), jnp.float32),
            pltpu.SemaphoreType.DMA,
        ],
    )
    return f(node_embeddings, padded_hyperedges, hyperedge_lengths)


# speedup vs baseline: 1.0005x; 1.0005x over previous
import functools

import jax
import jax.numpy as jnp
from jax import lax
from jax.experimental import pallas as pl
from jax.experimental.pallas import tpu as pltpu
from jax.experimental.pallas import tpu_sc as plsc

NUM_NODES = 100000
D = 128
B = 16384
M = 32

NC = 2
NS = 16
L = 16
NW = NC * NS
PER_W = B // NW
CH = 4
NCH = PER_W // CH
NG = D // L


def _splat(i32_scalar):
    return jnp.full((L,), i32_scalar, dtype=jnp.int32)


def _sc_body(table_hbm, idx_hbm, len_hbm, out_hbm,
             idx_raw, sel_v, len_v, recip_v, s2_v, rows_v, obuf, emb0_v,
             gsems, osems):
    wid = lax.axis_index("s") * NC + lax.axis_index("c")
    base = wid * PER_W

    pltpu.sync_copy(idx_hbm.at[pl.ds(base, PER_W)], idx_raw)
    pltpu.sync_copy(len_hbm.at[pl.ds(base, PER_W)], len_v)
    pltpu.sync_copy(table_hbm.at[0], emb0_v)

    iota = lax.iota(jnp.int32, L)

    def scales_body(g, _):
        l16 = len_v[pl.ds(g * L, L)]
        lenf = l16.astype(jnp.float32)
        rec = 1.0 / lenf
        recip_v[pl.ds(g * L, L)] = rec
        s2_v[pl.ds(g * L, L)] = (jnp.float32(M) - lenf) * rec
        return 0
    lax.fori_loop(0, PER_W // L, scales_body, 0, unroll=1)

    def mask_body(e, _):
        lsp = plsc.load_gather(len_v, [_splat(e)])
        iv0 = plsc.load_gather(idx_raw, [_splat(e), iota])
        iv1 = plsc.load_gather(idx_raw, [_splat(e), iota + L])
        s0 = jnp.where(iota < lsp, iv0, 0)
        s1 = jnp.where(iota + L < lsp, iv1, 0)
        row = e // CH
        col = (e % CH) * M
        plsc.store_scatter(sel_v, [_splat(row), col + iota], s0)
        plsc.store_scatter(sel_v, [_splat(row), col + L + iota], s1)
        return 0
    lax.fori_loop(0, PER_W, mask_body, 0, unroll=1)

    def start_gather(c, b):
        pltpu.async_copy(table_hbm.at[sel_v.at[c]], rows_v.at[b], gsems[b])

    def wait_gather(b):
        pltpu.make_async_copy(table_hbm.at[sel_v.at[0]], rows_v.at[b],
                              gsems[b]).wait()

    def start_owrite(c, b):
        pltpu.async_copy(obuf.at[b], out_hbm.at[pl.ds(base + c * CH, CH)],
                         osems[b])

    def wait_owrite(b):
        pltpu.make_async_copy(obuf.at[b], out_hbm.at[pl.ds(base, CH)],
                              osems[b]).wait()

    def compute(c, b):
        for el in range(CH):
            edge = c * CH + el
            rsp = plsc.load_gather(recip_v, [_splat(edge)])
            ssp = plsc.load_gather(s2_v, [_splat(edge)])
            r0 = el * M

            def acc_body(j, accs):
                return tuple(
                    accs[g] + rows_v[b, r0 + j, pl.ds(g * L, L)]
                    for g in range(NG)
                )
            init = tuple(rows_v[b, r0, pl.ds(g * L, L)] for g in range(NG))
            accs = lax.fori_loop(1, M, acc_body, init, unroll=4)
            for g in range(NG):
                obuf[b, el, pl.ds(g * L, L)] = (
                    accs[g] * rsp - emb0_v[pl.ds(g * L, L)] * ssp
                )

    start_gather(0, 0)

    def pair_body(c2, _):
        c = 2 * c2
        start_gather(c + 1, 1)
        wait_gather(0)

        @pl.when(c2 > 0)
        def _():
            wait_owrite(0)
        compute(c, 0)
        start_owrite(c, 0)

        @pl.when(c + 2 < NCH)
        def _():
            start_gather(c + 2, 0)
        wait_gather(1)

        @pl.when(c2 > 0)
        def _():
            wait_owrite(1)
        compute(c + 1, 1)
        start_owrite(c + 1, 1)
        return 0
    lax.fori_loop(0, NCH // 2, pair_body, 0, unroll=1)
    wait_owrite(0)
    wait_owrite(1)


def kernel(node_embeddings, padded_hyperedges, hyperedge_lengths):
    mesh = plsc.VectorSubcoreMesh(core_axis_name="c", subcore_axis_name="s")
    f = pl.kernel(
        _sc_body,
        out_type=jax.ShapeDtypeStruct((B, D), jnp.float32),
        mesh=mesh,
        compiler_params=pltpu.CompilerParams(needs_layout_passes=False),
        scratch_types=[
            pltpu.VMEM((PER_W, M), jnp.int32),
            pltpu.VMEM((NCH, CH * M), jnp.int32),
            pltpu.VMEM((PER_W,), jnp.int32),
            pltpu.VMEM((PER_W,), jnp.float32),
            pltpu.VMEM((PER_W,), jnp.float32),
            pltpu.VMEM((2, CH * M, D), jnp.float32),
            pltpu.VMEM((2, CH, D), jnp.float32),
            pltpu.VMEM((D,), jnp.float32),
            [pltpu.SemaphoreType.DMA, pltpu.SemaphoreType.DMA],
            [pltpu.SemaphoreType.DMA, pltpu.SemaphoreType.DMA],
        ],
    )
    return f(node_embeddings, padded_hyperedges, hyperedge_lengths)
